# Initial kernel scaffold; baseline (speedup 1.0000x reference)
import jax
import jax.numpy as jnp
from jax.experimental import pallas as pl


def kernel(m_atoms, m_bonds, m_edges, p_atoms, p_edges, W1m, b1m, W2m, b2m, Wom, bom, W1p, b1p, W2p, b2p, Wop, bop, Wf1, bf1, Wf2, bf2, Wout, bout):
    raise NotImplementedError("write your pallas kernel here")


# speedup vs baseline: 36.0553x; 36.0553x over previous
import jax
import jax.numpy as jnp
from jax import lax
from jax.experimental import pallas as pl
from jax.experimental.pallas import tpu as pltpu

_F32 = jnp.float32


def _body(ma_ref, mbt_ref, me_ref, pa_ref, pe_ref,
          W1m_ref, b1m_ref, W2m_ref, b2m_ref, Wom_ref, bom_ref,
          W1p_ref, b1p_ref, W2p_ref, b2p_ref, Wop_ref, bop_ref,
          Wf1_ref, bf1_ref, Wf2_ref, bf2_ref, Wout_ref, bout_ref,
          out_ref):
    dot = lambda a, b: jnp.dot(a, b, preferred_element_type=_F32)
    bf16 = jnp.bfloat16

    def gsum(A_bf, x):
        hi = x.astype(bf16)
        lo = (x - hi.astype(_F32)).astype(bf16)
        return (jnp.dot(A_bf, hi, preferred_element_type=_F32) +
                jnp.dot(A_bf, lo, preferred_element_type=_F32))

    ma = ma_ref[0]
    mbt = mbt_ref[0]
    me = me_ref[0]
    bs = jnp.sum(mbt, axis=0)

    iota_m = lax.broadcasted_iota(jnp.int32, (64, 64), 1)
    Am = jnp.zeros((64, 64), _F32)
    for k in range(6):
        Am = Am + (me[:, k:k + 1] == iota_m).astype(_F32)
    Am = Am.astype(bf16)

    x = ma + gsum(Am, ma)
    c1 = jax.nn.relu(dot(x, W1m_ref[...]) + b1m_ref[...])
    rbs = jax.nn.relu(bs)
    c1n = c1 + gsum(Am, c1)
    bsn = rbs + gsum(Am, rbs)
    c2 = jax.nn.relu(dot(c1n, W2m_ref[0:128, :]) +
                     dot(bsn, W2m_ref[128:134, :]) + b2m_ref[...])
    fpm = jnp.sum(jnp.tanh(dot(c2, Wom_ref[0:128, :]) +
                           dot(rbs, Wom_ref[128:134, :]) + bom_ref[...]),
                  axis=0, keepdims=True)

    pa = pa_ref[0]
    pe = pe_ref[0]

    iota_p = lax.broadcasted_iota(jnp.int32, (512, 512), 1)
    Ap = jnp.zeros((512, 512), _F32)
    for k in range(8):
        Ap = Ap + (pe[:, k:k + 1] == iota_p).astype(_F32)
    Ap = Ap.astype(bf16)

    g = jax.nn.relu(dot(pa + gsum(Ap, pa), W1p_ref[...]) + b1p_ref[...])
    g2 = jax.nn.relu(dot(g + gsum(Ap, g), W2p_ref[...]) + b2p_ref[...])
    fpp = jnp.sum(jnp.tanh(dot(g2, Wop_ref[...]) + bop_ref[...]),
                  axis=0, keepdims=True)

    f1 = dot(fpm, Wf1_ref[0:128, :]) + dot(fpp, Wf1_ref[128:256, :]) + bf1_ref[...]
    f2 = dot(f1, Wf2_ref[...]) + bf2_ref[...]
    o = jax.nn.sigmoid(dot(f2, Wout_ref[...]) + bout_ref[...])
    out_ref[...] = jnp.broadcast_to(o, (1, 1, 128))


def kernel(m_atoms, m_bonds, m_edges, p_atoms, p_edges,
           W1m, b1m, W2m, b2m, Wom, bom,
           W1p, b1p, W2p, b2p, Wop, bop,
           Wf1, bf1, Wf2, bf2, Wout, bout):
    B = m_atoms.shape[0]
    mbt = jnp.transpose(m_bonds, (0, 2, 1, 3))
    me = m_edges.astype(jnp.int32)
    pe = p_edges.astype(jnp.int32)
    row = lambda v: v.reshape(1, -1)

    def wspec(a):
        nd = a.ndim
        return pl.BlockSpec(a.shape, lambda b, _n=nd: (0,) * _n)

    weights = [W1m, row(b1m), W2m, row(b2m), Wom, row(bom),
               W1p, row(b1p), W2p, row(b2p), Wop, row(bop),
               Wf1, row(bf1), Wf2, row(bf2), Wout, row(bout)]

    out = pl.pallas_call(
        _body,
        grid=(B,),
        in_specs=[
            pl.BlockSpec((1, 64, 43), lambda b: (b, 0, 0)),
            pl.BlockSpec((1, 6, 64, 6), lambda b: (b, 0, 0, 0)),
            pl.BlockSpec((1, 64, 6), lambda b: (b, 0, 0)),
            pl.BlockSpec((1, 512, 480), lambda b: (b, 0, 0)),
            pl.BlockSpec((1, 512, 8), lambda b: (b, 0, 0)),
        ] + [wspec(w) for w in weights],
        out_specs=pl.BlockSpec((1, 1, 128), lambda b: (b, 0, 0)),
        out_shape=jax.ShapeDtypeStruct((B, 1, 128), _F32),
        compiler_params=pltpu.CompilerParams(
            dimension_semantics=("arbitrary",)),
    )(m_atoms, mbt, me, p_atoms, pe, *weights)
    return out[:, 0, :1]


# speedup vs baseline: 38.0197x; 1.0545x over previous
import jax
import jax.numpy as jnp
from jax import lax
from jax.experimental import pallas as pl
from jax.experimental.pallas import tpu as pltpu

_F32 = jnp.float32


def _body(ma_ref, mbt_ref, me_ref, pa_ref, pe_ref,
          W1m_ref, b1m_ref, W2m_ref, b2m_ref, Wom_ref, bom_ref,
          W1p_ref, b1p_ref, W2p_ref, b2p_ref, Wop_ref, bop_ref,
          Wf1_ref, bf1_ref, Wf2_ref, bf2_ref, Wout_ref, bout_ref,
          out_ref):
    dot = lambda a, b: jnp.dot(a, b, preferred_element_type=_F32)
    bf16 = jnp.bfloat16

    def gsum(A_bf, x):
        hi = x.astype(bf16)
        lo = (x - hi.astype(_F32)).astype(bf16)
        return (jnp.dot(A_bf, hi, preferred_element_type=_F32) +
                jnp.dot(A_bf, lo, preferred_element_type=_F32))

    ma = ma_ref[0]
    mbt = mbt_ref[0]
    me = me_ref[0]
    bs = jnp.sum(mbt, axis=0)

    one_m = jnp.ones((64, 64), bf16)
    zero_m = jnp.zeros((64, 64), bf16)
    me16 = me.astype(jnp.int16)
    iota_m = lax.broadcasted_iota(jnp.int16, (64, 64), 1)
    Am = jnp.zeros((64, 64), bf16)
    for k in range(6):
        Am = Am + jnp.where(me16[:, k:k + 1] == iota_m, one_m, zero_m)

    x = ma + gsum(Am, ma)
    c1 = jax.nn.relu(dot(x, W1m_ref[...]) + b1m_ref[...])
    rbs = jax.nn.relu(bs)
    c1n = c1 + gsum(Am, c1)
    bsn = rbs + gsum(Am, rbs)
    c2 = jax.nn.relu(dot(c1n, W2m_ref[0:128, :]) +
                     dot(bsn, W2m_ref[128:134, :]) + b2m_ref[...])
    fpm = jnp.sum(jnp.tanh(dot(c2, Wom_ref[0:128, :]) +
                           dot(rbs, Wom_ref[128:134, :]) + bom_ref[...]),
                  axis=0, keepdims=True)

    pa = pa_ref[0]
    pe = pe_ref[0]

    one_p = jnp.ones((512, 512), bf16)
    zero_p = jnp.zeros((512, 512), bf16)
    pe16 = pe.astype(jnp.int16)
    iota_p = lax.broadcasted_iota(jnp.int16, (512, 512), 1)
    Ap = jnp.zeros((512, 512), bf16)
    for k in range(8):
        Ap = Ap + jnp.where(pe16[:, k:k + 1] == iota_p, one_p, zero_p)

    g = jax.nn.relu(dot(pa + gsum(Ap, pa), W1p_ref[...]) + b1p_ref[...])
    g2 = jax.nn.relu(dot(g + gsum(Ap, g), W2p_ref[...]) + b2p_ref[...])
    fpp = jnp.sum(jnp.tanh(dot(g2, Wop_ref[...]) + bop_ref[...]),
                  axis=0, keepdims=True)

    f1 = dot(fpm, Wf1_ref[0:128, :]) + dot(fpp, Wf1_ref[128:256, :]) + bf1_ref[...]
    f2 = dot(f1, Wf2_ref[...]) + bf2_ref[...]
    o = jax.nn.sigmoid(dot(f2, Wout_ref[...]) + bout_ref[...])
    out_ref[...] = jnp.broadcast_to(o, (1, 1, 128))


def kernel(m_atoms, m_bonds, m_edges, p_atoms, p_edges,
           W1m, b1m, W2m, b2m, Wom, bom,
           W1p, b1p, W2p, b2p, Wop, bop,
           Wf1, bf1, Wf2, bf2, Wout, bout):
    B = m_atoms.shape[0]
    mbt = jnp.transpose(m_bonds, (0, 2, 1, 3))
    me = m_edges.astype(jnp.int32)
    pe = p_edges.astype(jnp.int32)
    row = lambda v: v.reshape(1, -1)

    def wspec(a):
        nd = a.ndim
        return pl.BlockSpec(a.shape, lambda b, _n=nd: (0,) * _n)

    weights = [W1m, row(b1m), W2m, row(b2m), Wom, row(bom),
               W1p, row(b1p), W2p, row(b2p), Wop, row(bop),
               Wf1, row(bf1), Wf2, row(bf2), Wout, row(bout)]

    out = pl.pallas_call(
        _body,
        grid=(B,),
        in_specs=[
            pl.BlockSpec((1, 64, 43), lambda b: (b, 0, 0)),
            pl.BlockSpec((1, 6, 64, 6), lambda b: (b, 0, 0, 0)),
            pl.BlockSpec((1, 64, 6), lambda b: (b, 0, 0)),
            pl.BlockSpec((1, 512, 480), lambda b: (b, 0, 0)),
            pl.BlockSpec((1, 512, 8), lambda b: (b, 0, 0)),
        ] + [wspec(w) for w in weights],
        out_specs=pl.BlockSpec((1, 1, 128), lambda b: (b, 0, 0)),
        out_shape=jax.ShapeDtypeStruct((B, 1, 128), _F32),
        compiler_params=pltpu.CompilerParams(
            dimension_semantics=("arbitrary",)),
    )(m_atoms, mbt, me, p_atoms, pe, *weights)
    return out[:, 0, :1]


# speedup vs baseline: 41.0110x; 1.0787x over previous
import jax
import jax.numpy as jnp
from jax import lax
from jax.experimental import pallas as pl
from jax.experimental.pallas import tpu as pltpu

_F32 = jnp.float32
_BP = 4


def _one(j, ma_ref, mbt_ref, me_ref, pa_ref, pe_ref,
         W1m_ref, b1m_ref, W2m_ref, b2m_ref, Wom_ref, bom_ref,
         W1p_ref, b1p_ref, W2p_ref, b2p_ref, Wop_ref, bop_ref,
         Wf1_ref, bf1_ref, Wf2_ref, bf2_ref, Wout_ref, bout_ref,
         out_ref):
    dot = lambda a, b: jnp.dot(a, b, preferred_element_type=_F32)
    bf16 = jnp.bfloat16

    def gsum(A_bf, x):
        hi = x.astype(bf16)
        lo = (x - hi.astype(_F32)).astype(bf16)
        return (jnp.dot(A_bf, hi, preferred_element_type=_F32) +
                jnp.dot(A_bf, lo, preferred_element_type=_F32))

    one = jnp.bfloat16(1)
    zero = jnp.bfloat16(0)

    ma = ma_ref[j]
    mbt = mbt_ref[j]
    me = me_ref[j]
    bs = jnp.sum(mbt, axis=0)

    me16 = me.astype(jnp.int16)
    iota_m = lax.broadcasted_iota(jnp.int16, (64, 64), 1)
    Am = jnp.zeros((64, 64), bf16)
    for k in range(6):
        Am = Am + jnp.where(me16[:, k:k + 1] == iota_m, one, zero)

    x = ma + gsum(Am, ma)
    c1 = jax.nn.relu(dot(x, W1m_ref[...]) + b1m_ref[...])
    rbs = jax.nn.relu(bs)
    c1n = c1 + gsum(Am, c1)
    bsn = rbs + gsum(Am, rbs)
    c2 = jax.nn.relu(dot(c1n, W2m_ref[0:128, :]) +
                     dot(bsn, W2m_ref[128:134, :]) + b2m_ref[...])
    fpm = jnp.sum(jnp.tanh(dot(c2, Wom_ref[0:128, :]) +
                           dot(rbs, Wom_ref[128:134, :]) + bom_ref[...]),
                  axis=0, keepdims=True)

    pa = pa_ref[j]
    pe = pe_ref[j]

    pe16 = pe.astype(jnp.int16)
    iota_p = lax.broadcasted_iota(jnp.int16, (512, 512), 1)
    Ap = jnp.zeros((512, 512), bf16)
    for k in range(8):
        Ap = Ap + jnp.where(pe16[:, k:k + 1] == iota_p, one, zero)

    g = jax.nn.relu(dot(pa + gsum(Ap, pa), W1p_ref[...]) + b1p_ref[...])
    g2 = jax.nn.relu(dot(g + gsum(Ap, g), W2p_ref[...]) + b2p_ref[...])
    fpp = jnp.sum(jnp.tanh(dot(g2, Wop_ref[...]) + bop_ref[...]),
                  axis=0, keepdims=True)

    f1 = dot(fpm, Wf1_ref[0:128, :]) + dot(fpp, Wf1_ref[128:256, :]) + bf1_ref[...]
    f2 = dot(f1, Wf2_ref[...]) + bf2_ref[...]
    o = jax.nn.sigmoid(dot(f2, Wout_ref[...]) + bout_ref[...])
    out_ref[j] = jnp.broadcast_to(o, (1, 128))


def _body(*refs):
    for j in range(_BP):
        _one(j, *refs)


def kernel(m_atoms, m_bonds, m_edges, p_atoms, p_edges,
           W1m, b1m, W2m, b2m, Wom, bom,
           W1p, b1p, W2p, b2p, Wop, bop,
           Wf1, bf1, Wf2, bf2, Wout, bout):
    B = m_atoms.shape[0]
    mbt = jnp.transpose(m_bonds, (0, 2, 1, 3))
    me = m_edges.astype(jnp.int32)
    pe = p_edges.astype(jnp.int32)
    row = lambda v: v.reshape(1, -1)

    def wspec(a):
        nd = a.ndim
        return pl.BlockSpec(a.shape, lambda b, _n=nd: (0,) * _n)

    weights = [W1m, row(b1m), W2m, row(b2m), Wom, row(bom),
               W1p, row(b1p), W2p, row(b2p), Wop, row(bop),
               Wf1, row(bf1), Wf2, row(bf2), Wout, row(bout)]

    out = pl.pallas_call(
        _body,
        grid=(B // _BP,),
        in_specs=[
            pl.BlockSpec((_BP, 64, 43), lambda b: (b, 0, 0)),
            pl.BlockSpec((_BP, 6, 64, 6), lambda b: (b, 0, 0, 0)),
            pl.BlockSpec((_BP, 64, 6), lambda b: (b, 0, 0)),
            pl.BlockSpec((_BP, 512, 480), lambda b: (b, 0, 0)),
            pl.BlockSpec((_BP, 512, 8), lambda b: (b, 0, 0)),
        ] + [wspec(w) for w in weights],
        out_specs=pl.BlockSpec((_BP, 1, 128), lambda b: (b, 0, 0)),
        out_shape=jax.ShapeDtypeStruct((B, 1, 128), _F32),
        compiler_params=pltpu.CompilerParams(
            dimension_semantics=("arbitrary",)),
    )(m_atoms, mbt, me, p_atoms, pe, *weights)
    return out[:, 0, :1]
